# baseline (speedup 1.0000x reference)
import functools
import jax
import jax.numpy as jnp
from jax import lax
from jax.experimental import pallas as pl
from jax.experimental.pallas import tpu as pltpu
from jax.experimental.pallas import tpu_sc as plsc

D_MODEL = 256
SCALE = 16.0
TABLE_ROWS = 68

NC = 2
NS = 16
NW = NC * NS
CHUNK = 128
L = 16


@functools.cache
def _make_lookup(n_rows):
    rows_per_w = n_rows // NW
    n_chunks = rows_per_w // CHUNK
    assert n_chunks % 2 == 0
    mesh = plsc.VectorSubcoreMesh(
        core_axis_name="c", subcore_axis_name="s",
        num_cores=NC, num_subcores=NS,
    )

    @functools.partial(
        pl.kernel,
        out_type=jax.ShapeDtypeStruct((n_rows, D_MODEL), jnp.float32),
        mesh=mesh,
        scratch_types=[
            pltpu.VMEM((TABLE_ROWS * D_MODEL,), jnp.float32),
            pltpu.VMEM((n_chunks, CHUNK), jnp.int32),
            pltpu.VMEM((2, CHUNK, D_MODEL), jnp.float32),
            pltpu.SemaphoreType.DMA,
            pltpu.SemaphoreType.DMA,
        ],
    )
    def lookup_kernel(table_hbm, idx_hbm, out_hbm, tbl_v, idx_v, rows_v,
                      s_sem0, s_sem1):
        wid = lax.axis_index("s") * NC + lax.axis_index("c")
        base = wid * rows_per_w
        pltpu.sync_copy(table_hbm, tbl_v)
        pltpu.sync_copy(idx_hbm.at[wid], idx_v)

        def scale_row(r, carry):
            off = r * D_MODEL
            vals = [tbl_v[pl.ds(off + k * L, L)]
                    for k in range(D_MODEL // L)]
            for k, v in enumerate(vals):
                tbl_v[pl.ds(off + k * L, L)] = v * SCALE
            return carry

        lax.fori_loop(0, TABLE_ROWS, scale_row, 0)

        s_sems = (s_sem0, s_sem1)
        kv = D_MODEL // L

        def assemble(j, buf):
            def pair_body(h, carry):
                prev = None
                prev_dst = 0
                for gg in range(4):
                    g = h * 4 + gg
                    iv = idx_v[j, pl.ds(g * L, L)] * D_MODEL
                    for l in range(L):
                        src = iv[l]
                        vals = []
                        for k in range(kv):
                            if prev is not None:
                                rows_v[buf, prev_dst, pl.ds(k * L, L)] = (
                                    prev[k])
                            vals.append(tbl_v[pl.ds(src + k * L, L)])
                        prev = vals
                        prev_dst = g * L + l
                for k in range(kv):
                    rows_v[buf, prev_dst, pl.ds(k * L, L)] = prev[k]
                return carry
            lax.fori_loop(0, CHUNK // (4 * L), pair_body, 0)

        def s_copy(j, buf):
            return pltpu.make_async_copy(
                rows_v.at[buf],
                out_hbm.at[pl.ds(base + j * CHUNK, CHUNK)], s_sems[buf])

        def body(i, carry):
            j0 = i * 2
            j1 = j0 + 1

            @pl.when(i > 0)
            def _():
                s_copy(j0 - 2, 0).wait()

            assemble(j0, 0)
            s_copy(j0, 0).start()

            @pl.when(i > 0)
            def _():
                s_copy(j1 - 2, 1).wait()

            assemble(j1, 1)
            s_copy(j1, 1).start()
            return carry

        lax.fori_loop(0, n_chunks // 2, body, 0)
        s_copy(n_chunks - 2, 0).wait()
        s_copy(n_chunks - 1, 1).wait()

    return lookup_kernel


def kernel(x, table):
    B, S = x.shape
    n = B * S
    idx = x.reshape(NW, n // NW // CHUNK, CHUNK).astype(jnp.int32)
    out = _make_lookup(n)(table.reshape(TABLE_ROWS * D_MODEL), idx)
    return out.reshape(B, S, D_MODEL)


# speedup vs baseline: 1.6238x; 1.6238x over previous
import functools
import jax
import jax.numpy as jnp
from jax import lax
from jax.experimental import pallas as pl
from jax.experimental.pallas import tpu as pltpu
from jax.experimental.pallas import tpu_sc as plsc

D_MODEL = 256
SCALE = 16.0
TABLE_ROWS = 68

NC = 2
NS = 16
NW = NC * NS
CHUNK = 128
L = 16


@functools.cache
def _make_lookup(n_rows):
    rows_per_w = n_rows // NW
    n_chunks = rows_per_w // CHUNK
    assert n_chunks % 2 == 0
    mesh = plsc.VectorSubcoreMesh(
        core_axis_name="c", subcore_axis_name="s",
        num_cores=NC, num_subcores=NS,
    )

    @functools.partial(
        pl.kernel,
        out_type=jax.ShapeDtypeStruct((n_rows, D_MODEL), jnp.float32),
        mesh=mesh,
        scratch_types=[
            pltpu.VMEM((TABLE_ROWS * D_MODEL,), jnp.float32),
            pltpu.VMEM((n_chunks, CHUNK), jnp.int32),
            pltpu.VMEM((2, CHUNK, D_MODEL), jnp.float32),
            pltpu.SemaphoreType.DMA,
            pltpu.SemaphoreType.DMA,
        ],
    )
    def lookup_kernel(table_hbm, idx_hbm, out_hbm, tbl_v, idx_v, rows_v,
                      s_sem0, s_sem1):
        wid = lax.axis_index("s") * NC + lax.axis_index("c")
        base = wid * rows_per_w
        pltpu.sync_copy(table_hbm, tbl_v)
        pltpu.sync_copy(idx_hbm.at[wid], idx_v)

        def scale_row(r, carry):
            off = r * D_MODEL
            vals = [tbl_v[pl.ds(off + k * L, L)]
                    for k in range(D_MODEL // L)]
            for k, v in enumerate(vals):
                tbl_v[pl.ds(off + k * L, L)] = v * SCALE
            return carry

        lax.fori_loop(0, TABLE_ROWS, scale_row, 0)

        s_sems = (s_sem0, s_sem1)
        kv = D_MODEL // L

        def assemble(j, buf):
            def pair_body(h, carry):
                prev = None
                prev_dst = 0
                for gg in range(2):
                    g = h * 2 + gg
                    iv = idx_v[j, pl.ds(g * L, L)] * D_MODEL
                    for l in range(L):
                        src = iv[l]
                        vals = []
                        for k in range(kv):
                            if prev is not None:
                                rows_v[buf, prev_dst, pl.ds(k * L, L)] = (
                                    prev[k])
                            vals.append(tbl_v[pl.ds(src + k * L, L)])
                        prev = vals
                        prev_dst = g * L + l
                for k in range(kv):
                    rows_v[buf, prev_dst, pl.ds(k * L, L)] = prev[k]
                return carry
            lax.fori_loop(0, CHUNK // (2 * L), pair_body, 0)

        def s_copy(j, buf):
            return pltpu.make_async_copy(
                rows_v.at[buf],
                out_hbm.at[pl.ds(base + j * CHUNK, CHUNK)], s_sems[buf])

        def body(i, carry):
            j0 = i * 2
            j1 = j0 + 1

            @pl.when(i > 0)
            def _():
                s_copy(j0 - 2, 0).wait()

            assemble(j0, 0)
            s_copy(j0, 0).start()

            @pl.when(i > 0)
            def _():
                s_copy(j1 - 2, 1).wait()

            assemble(j1, 1)
            s_copy(j1, 1).start()
            return carry

        lax.fori_loop(0, n_chunks // 2, body, 0)
        s_copy(n_chunks - 2, 0).wait()
        s_copy(n_chunks - 1, 1).wait()

    return lookup_kernel


def kernel(x, table):
    B, S = x.shape
    n = B * S
    idx = x.reshape(NW, n // NW // CHUNK, CHUNK).astype(jnp.int32)
    out = _make_lookup(n)(table.reshape(TABLE_ROWS * D_MODEL), idx)
    return out.reshape(B, S, D_MODEL)
